# baseline (speedup 1.0000x reference)
import jax
import jax.numpy as jnp
from jax import lax
from jax.experimental import pallas as pl
from jax.experimental.pallas import tpu as pltpu

_H = _W = 32
_CIN, _K, _OC1, _OC2, _FC1 = 3, 5, 5, 10, 32
_OH1, _P1H, _OH2, _P2H = 28, 14, 10, 5

_B = 512
_NPIX = _CIN * _H * _W
_XROWS = 3136
_K1 = 512
_M1 = _OC1 * 32
_P1ROWS = 9 * 80 + _K1
_K2 = 512
_M2 = _OC2 * 16
_P2ROWS = 512


def _body(x_ref, w1m, b1m, w2m, b2m, w1p, b1fc, w2fc, b2fc, out_ref,
          xs, p1c, p2c):
    f32 = jnp.float32

    def xpose(j):
        ic, h0 = j // 8, (j % 8) * 4
        xt = x_ref[:, j * 128:(j + 1) * 128].T
        for t in range(4):
            xs[pl.ds((h0 + t) * 96 + ic * 32, 32), :] = xt[t * 32:(t + 1) * 32]

    xs[pl.ds(_NPIX, _XROWS - _NPIX), :] = jnp.zeros((_XROWS - _NPIX, _B), f32)
    p1c[pl.ds(14 * 80, _P1ROWS - 14 * 80), :] = jnp.zeros(
        (_P1ROWS - 14 * 80, _B), f32)
    p2c[pl.ds(400, _P2ROWS - 400), :] = jnp.zeros((_P2ROWS - 400, _B), f32)

    for j in (0, 1, 2, 8, 9, 10, 16, 17, 18):
        xpose(j)

    for ph in range(_P1H):
        if ph < 5:
            for ic in range(_CIN):
                xpose(ic * 8 + 3 + ph)
        r0 = 2 * ph * 96
        r1 = r0 + 96
        a = jnp.dot(w1m[...], xs[pl.ds(r0, _K1), :],
                    preferred_element_type=f32).reshape(_OC1, 2, 16, _B)
        b = jnp.dot(w1m[...], xs[pl.ds(r1, _K1), :],
                    preferred_element_type=f32).reshape(_OC1, 2, 16, _B)
        m = jnp.maximum(jnp.maximum(a[:, 0], a[:, 1]),
                        jnp.maximum(b[:, 0], b[:, 1])).reshape(80, _B)
        p1c[pl.ds(ph * 80, 80), :] = jnp.maximum(m + b1m[...], 0.0)

    for q in range(_P2H):
        r0 = 2 * q * 80
        r1 = r0 + 80
        a = jnp.dot(w2m[...], p1c[pl.ds(r0, _K2), :],
                    preferred_element_type=f32).reshape(_OC2, 2, 8, _B)
        b = jnp.dot(w2m[...], p1c[pl.ds(r1, _K2), :],
                    preferred_element_type=f32).reshape(_OC2, 2, 8, _B)
        m = jnp.maximum(jnp.maximum(a[:, 0], a[:, 1]),
                        jnp.maximum(b[:, 0], b[:, 1])).reshape(80, _B)
        p2c[pl.ds(q * 80, 80), :] = jnp.maximum(m + b2m[...], 0.0)

    hfc = jnp.maximum(
        jnp.dot(w1p[...], p2c[...], preferred_element_type=f32) + b1fc[...],
        0.0)
    out_ref[...] = jnp.dot(w2fc[...], hfc, preferred_element_type=f32) \
        + b2fc[...]


def _stride2_toeplitz(w, rows, width):
    pitch = width + 2
    lead = w.shape[:-1]
    p = jnp.pad(w, [(0, 0)] * len(lead) + [(0, pitch - w.shape[-1])])
    p = jnp.broadcast_to(p[..., None, :], lead + (rows, pitch))
    p = p.reshape(lead + (rows * pitch,))[..., :rows * width]
    return p.reshape(lead + (rows, width))


def _conv1_matrix(w1):
    w1r = w1.reshape(_OC1, _CIN, _K, _K)
    evn = _stride2_toeplitz(w1r, 16, 32)
    odd = _stride2_toeplitz(jnp.pad(w1r, ((0, 0),) * 3 + ((1, 0),))[..., :_K + 1],
                            16, 32)
    wm = jnp.concatenate([evn, odd], axis=3)
    wm = wm.transpose(0, 3, 2, 1, 4).reshape(_M1, 480)
    return jnp.pad(wm, ((0, 0), (0, _K1 - 480)))


def _conv2_matrix(w2):
    w2r = w2.reshape(_OC2, _OC1, _K, _K)
    evn = _stride2_toeplitz(w2r, 8, 16)
    odd = _stride2_toeplitz(jnp.pad(w2r, ((0, 0),) * 3 + ((1, 0),))[..., :_K + 1],
                            8, 16)
    wm = jnp.concatenate([evn, odd], axis=3)
    wm = wm.transpose(0, 3, 2, 1, 4).reshape(_M2, 400)
    return jnp.pad(wm, ((0, 0), (0, _K2 - 400)))


def _fc1_matrix(w1u):
    w3 = jnp.pad(w1u.reshape(_FC1, _OC2, 280), ((0, 0), (0, 0), (0, 40)))
    w4 = w3.reshape(_FC1, _OC2, _P2H, 64)[:, :, :, 0:20:4]
    w4 = jnp.pad(w4.transpose(0, 2, 1, 3), ((0, 0),) * 3 + ((0, 3),))
    return jnp.pad(w4.reshape(_FC1, 400), ((0, 0), (0, _P2ROWS - 400)))


def kernel(w1, b1, w2, b2, w1u, b1fc, w2fc, b2fc, x_nchw):
    n = x_nchw.shape[0]
    g = pl.cdiv(n, _B)
    npad = g * _B

    x = x_nchw.astype(jnp.float32)
    if npad != n:
        x = jnp.pad(x, ((0, npad - n), (0, 0), (0, 0), (0, 0)))
    x2d = x.reshape(npad, _NPIX)

    w1m = _conv1_matrix(w1)
    w2m = _conv2_matrix(w2)
    w1p = _fc1_matrix(w1u)
    b1m = jnp.repeat(b1.astype(jnp.float32), 16)[:, None]
    b2m = jnp.repeat(b2.astype(jnp.float32), 8)[:, None]

    fix = lambda s: pl.BlockSpec(s, lambda b: (0,) * len(s))

    out = pl.pallas_call(
        _body,
        out_shape=jax.ShapeDtypeStruct((1, npad), jnp.float32),
        grid=(g,),
        in_specs=[
            pl.BlockSpec((_B, _NPIX), lambda b: (b, 0)),
            fix((_M1, _K1)), fix((80, 1)),
            fix((_M2, _K2)), fix((80, 1)),
            fix((_FC1, _P2ROWS)), fix((_FC1, 1)),
            fix((1, _FC1)), fix((1, 1)),
        ],
        out_specs=pl.BlockSpec((1, _B), lambda b: (0, b)),
        scratch_shapes=[
            pltpu.VMEM((_XROWS, _B), jnp.float32),
            pltpu.VMEM((_P1ROWS, _B), jnp.float32),
            pltpu.VMEM((_P2ROWS, _B), jnp.float32),
        ],
        compiler_params=pltpu.CompilerParams(
            dimension_semantics=("parallel",),
            vmem_limit_bytes=64 * 1024 * 1024),
    )(x2d, w1m, b1m, w2m, b2m, w1p, b1fc, w2fc, b2fc)

    return out[0, :n]
